# baseline (speedup 1.0000x reference)
import dataclasses
import functools

import jax
import jax.numpy as jnp
from jax import lax
from jax.experimental import pallas as pl
from jax.experimental.pallas import tpu as pltpu
from jax.experimental.pallas import tpu_sc as plsc

NC = 2
NS = 16
L = 16
NW = NC * NS
BATCH = 16384
D = 64
B_PER_W = BATCH // NW
CHUNKS = D // L
GROUPS = B_PER_W // L

_sc_mesh = plsc.VectorSubcoreMesh(
    core_axis_name="c", subcore_axis_name="s", num_cores=NC, num_subcores=NS
)

_sc_params = pltpu.CompilerParams(
    needs_layout_passes=False, use_tc_tiling_on_sc=False
)


def _sc_dist2(idx_u, idx_v, table):
    @functools.partial(
        pl.kernel,
        out_type=jax.ShapeDtypeStruct((BATCH,), jnp.float32),
        mesh=_sc_mesh,
        scratch_types=[
            pltpu.VMEM((B_PER_W,), jnp.int32),
            pltpu.VMEM((B_PER_W,), jnp.int32),
            pltpu.VMEM((B_PER_W, D), jnp.float32),
            pltpu.VMEM((B_PER_W, D), jnp.float32),
            pltpu.VMEM((B_PER_W,), jnp.float32),
            pltpu.SemaphoreType.DMA,
        ],
        compiler_params=_sc_params,
    )
    def k(table_hbm, iu_hbm, iv_hbm, out_hbm, iu_v, iv_v, u_v, v_v, d2_v, sem):
        wid = lax.axis_index("s") * NC + lax.axis_index("c")
        base = wid * B_PER_W
        pltpu.sync_copy(iu_hbm.at[pl.ds(base, B_PER_W)], iu_v)
        pltpu.sync_copy(iv_hbm.at[pl.ds(base, B_PER_W)], iv_v)
        cu = pltpu.async_copy(table_hbm.at[iu_v], u_v, sem)
        cv = pltpu.async_copy(table_hbm.at[iv_v], v_v, sem)
        cu.wait()
        cv.wait()

        lanes = lax.iota(jnp.int32, L)

        @pl.loop(0, GROUPS)
        def _(g):
            vec = jnp.zeros((L,), jnp.float32)
            for j in range(L):
                p = g * L + j
                acc = jnp.zeros((L,), jnp.float32)
                for c in range(CHUNKS):
                    du = u_v[p, pl.ds(c * L, L)] - v_v[p, pl.ds(c * L, L)]
                    acc = acc + du * du
                vec = jnp.where(lanes == j, jnp.sum(acc), vec)
            d2_v[pl.ds(g * L, L)] = vec

        pltpu.sync_copy(d2_v, out_hbm.at[pl.ds(base, B_PER_W)])

    return k(table, idx_u, idx_v)


def _tc_loss_body(d2_ref, lab_ref, bg_ref, out_ref):
    beta = bg_ref[0, 0]
    gamma = bg_ref[0, 1]
    dist = jnp.sqrt(d2_ref[...] + 1e-12)
    s = beta * dist - gamma
    signed = jnp.where(lab_ref[...] == 1.0, s, -s)
    out_ref[...] = jnp.logaddexp(0.0, signed)


def _tc_loss(d2, labels_f32, bg):
    r, c = 128, BATCH // 128
    out = pl.pallas_call(
        _tc_loss_body,
        out_shape=jax.ShapeDtypeStruct((r, c), jnp.float32),
    )(d2.reshape(r, c), labels_f32.reshape(r, c), bg)
    return out.reshape(BATCH)


@jax.jit
def kernel(pairs, labels, table, beta, gamma):
    idx_u = pairs[:, 0]
    idx_v = pairs[:, 1]
    d2 = _sc_dist2(idx_u, idx_v, table)
    bg = jnp.stack([beta, gamma]).reshape(1, 2).astype(jnp.float32)
    return _tc_loss(d2, labels.astype(jnp.float32), bg)


# speedup vs baseline: 2.1433x; 2.1433x over previous
import functools

import jax
import jax.numpy as jnp
from jax import lax
from jax.experimental import pallas as pl
from jax.experimental.pallas import tpu as pltpu
from jax.experimental.pallas import tpu_sc as plsc

NC = 2
NS = 16
L = 16
NW = NC * NS
BATCH = 16384
D = 64
R = 8
B_PER_W = BATCH // NW
W = 16
WINDOWS = B_PER_W // W

_sc_mesh = plsc.VectorSubcoreMesh(
    core_axis_name="c", subcore_axis_name="s", num_cores=NC, num_subcores=NS
)

_sc_params = pltpu.CompilerParams(needs_layout_passes=False)


def _sc_dist2(idx_u, idx_v, table3):
    @functools.partial(
        pl.kernel,
        out_type=jax.ShapeDtypeStruct((BATCH,), jnp.float32),
        mesh=_sc_mesh,
        scratch_types=[
            pltpu.VMEM((B_PER_W,), jnp.int32),
            pltpu.VMEM((B_PER_W,), jnp.int32),
            pltpu.VMEM((2, W, R, D), jnp.float32),
            pltpu.VMEM((2, W, R, D), jnp.float32),
            pltpu.VMEM((B_PER_W,), jnp.float32),
            pltpu.SemaphoreType.DMA,
            pltpu.SemaphoreType.DMA,
            pltpu.SemaphoreType.DMA,
        ],
        compiler_params=_sc_params,
    )
    def k(t3_hbm, iu_hbm, iv_hbm, out_hbm, iu_v, iv_v,
          ubuf, vbuf, d2_v, sem0, sem1, isem):
        wid = lax.axis_index("s") * NC + lax.axis_index("c")
        base = wid * B_PER_W
        pltpu.async_copy(iu_hbm.at[pl.ds(base, B_PER_W)], iu_v, isem).wait()
        pltpu.async_copy(iv_hbm.at[pl.ds(base, B_PER_W)], iv_v, isem).wait()

        sems = (sem0, sem1)
        lanes = lax.iota(jnp.int32, L)

        zeros = jnp.zeros((L,), jnp.int32)

        def fire(w, slot, sem):
            tu = lax.shift_right_logical(iu_v[pl.ds(w * W, W)], 3)
            tv = lax.shift_right_logical(iv_v[pl.ds(w * W, W)], 3)
            for j in range(W):
                su = jnp.sum(jnp.where(lanes == j, tu, zeros))
                sv = jnp.sum(jnp.where(lanes == j, tv, zeros))
                pltpu.async_copy(t3_hbm.at[su], ubuf.at[slot, j], sem)
                pltpu.async_copy(t3_hbm.at[sv], vbuf.at[slot, j], sem)

        def drain(slot, sem):
            pltpu.make_async_copy(
                t3_hbm.at[pl.ds(0, W)], ubuf.at[slot], sem).wait()
            pltpu.make_async_copy(
                t3_hbm.at[pl.ds(0, W)], vbuf.at[slot], sem).wait()

        def compute(w, slot):
            ru = iu_v[pl.ds(w * W, W)] & (R - 1)
            rv = iv_v[pl.ds(w * W, W)] & (R - 1)
            acc = jnp.zeros((L,), jnp.float32)
            for d in range(D):
                dsplat = jnp.full((L,), d, jnp.int32)
                du = plsc.load_gather(ubuf.at[slot], [lanes, ru, dsplat])
                dv = plsc.load_gather(vbuf.at[slot], [lanes, rv, dsplat])
                diff = du - dv
                acc = acc + diff * diff
            d2_v[pl.ds(w * W, W)] = acc

        fire(0, 0, sem0)
        fire(1, 1, sem1)

        @pl.loop(0, WINDOWS, step=2)
        def _(w):
            for b in range(2):
                drain(b, sems[b])
                compute(w + b, b)

                @pl.when(w + 2 + b < WINDOWS)
                def _():
                    fire(w + 2 + b, b, sems[b])

        pltpu.sync_copy(d2_v, out_hbm.at[pl.ds(base, B_PER_W)])

    return k(table3, idx_u, idx_v)


def _tc_loss_body(d2_ref, lab_ref, bg_ref, out_ref):
    beta = bg_ref[0, 0]
    gamma = bg_ref[0, 1]
    dist = jnp.sqrt(d2_ref[...] + 1e-12)
    s = beta * dist - gamma
    signed = jnp.where(lab_ref[...] == 1.0, s, -s)
    out_ref[...] = jnp.logaddexp(0.0, signed)


def _tc_loss(d2, labels_f32, bg):
    r, c = 128, BATCH // 128
    out = pl.pallas_call(
        _tc_loss_body,
        out_shape=jax.ShapeDtypeStruct((r, c), jnp.float32),
    )(d2.reshape(r, c), labels_f32.reshape(r, c), bg)
    return out.reshape(BATCH)


@jax.jit
def kernel(pairs, labels, table, beta, gamma):
    idx_u = pairs[:, 0]
    idx_v = pairs[:, 1]
    table3 = table.reshape(1000000 // R, R, D)
    d2 = _sc_dist2(idx_u, idx_v, table3)
    bg = jnp.stack([beta, gamma]).reshape(1, 2).astype(jnp.float32)
    return _tc_loss(d2, labels.astype(jnp.float32), bg)
